# baseline (speedup 1.0000x reference)
import functools

import jax
import jax.numpy as jnp
from jax import lax
from jax.experimental import pallas as pl
from jax.experimental.pallas import tpu as pltpu
from jax.experimental.pallas import tpu_sc as plsc

_NC = 2
_NS = 16
_NW = _NC * _NS
_CHUNK = 128
_LANES = 16


@functools.lru_cache(maxsize=None)
def _make_spmm(M, M_pad, CH, nchunk):
    rows_per_tile = M_pad // _NS

    mesh = plsc.VectorSubcoreMesh(core_axis_name="c", subcore_axis_name="s")

    @functools.partial(
        pl.kernel,
        out_type=jax.ShapeDtypeStruct((_NC, 2, M_pad, CH), jnp.float32),
        mesh=mesh,
        scratch_types=[
            pltpu.VMEM((nchunk, _CHUNK), jnp.int32),
            pltpu.VMEM((nchunk, _CHUNK), jnp.int32),
            pltpu.VMEM((nchunk, _CHUNK), jnp.float32),
            pltpu.VMEM((_CHUNK, CH), jnp.float32),
            pltpu.VMEM((_CHUNK, CH), jnp.float32),
            pltpu.VMEM_SHARED((M_pad, CH), jnp.float32),
            pltpu.SemaphoreType.DMA,
            pltpu.SemaphoreType.DMA,
        ],
        compiler_params=pltpu.CompilerParams(use_tc_tiling_on_sc=False),
    )
    def spmm(x0_hbm, x1_hbm, rows_hbm, cols_hbm, vals_hbm, zinit_hbm,
             out_hbm, rows_v, cols_v, vals_v, gbuf0, gbuf1, yacc,
             sem0, sem1):
        c = lax.axis_index("c")
        s = lax.axis_index("s")
        wid = c * _NS + s

        pltpu.sync_copy(rows_hbm.at[wid], rows_v)
        pltpu.sync_copy(cols_hbm.at[wid], cols_v)
        pltpu.sync_copy(vals_hbm.at[wid], vals_v)
        base = pl.multiple_of(s * rows_per_tile, 8)

        assert nchunk % 2 == 0
        niter = nchunk // 2

        for h, x_hbm in enumerate((x0_hbm, x1_hbm)):
            pltpu.sync_copy(zinit_hbm, yacc.at[pl.ds(base, rows_per_tile)])
            plsc.subcore_barrier()

            def start(j, buf, sem):
                return pltpu.async_copy(x_hbm.at[cols_v.at[j]], buf, sem)

            def process(j, buf):
                for eg in range(_CHUNK // _LANES):
                    v16 = vals_v[j, pl.ds(eg * _LANES, _LANES)]
                    for l in range(_LANES):
                        e = eg * _LANES + l
                        v = jnp.broadcast_to(v16[l], (_LANES,))
                        for g in range(CH // _LANES):
                            sl = pl.ds(g * _LANES, _LANES)
                            buf[e, sl] = buf[e, sl] * v
                pltpu.sync_copy(buf, yacc.at[rows_v.at[j]], add=True)

            start(0, gbuf0, sem0)

            def pair_body(i, carry):
                j0 = 2 * i
                start(j0 + 1, gbuf1, sem1)
                pltpu.make_async_copy(x_hbm.at[cols_v.at[j0]], gbuf0,
                                      sem0).wait()
                process(j0, gbuf0)
                start(j0 + 2, gbuf0, sem0)
                pltpu.make_async_copy(x_hbm.at[cols_v.at[j0 + 1]], gbuf1,
                                      sem1).wait()
                process(j0 + 1, gbuf1)
                return carry

            lax.fori_loop(0, niter - 1, pair_body, 0)
            j0 = nchunk - 2
            start(j0 + 1, gbuf1, sem1)
            pltpu.make_async_copy(x_hbm.at[cols_v.at[j0]], gbuf0,
                                  sem0).wait()
            process(j0, gbuf0)
            pltpu.make_async_copy(x_hbm.at[cols_v.at[j0 + 1]], gbuf1,
                                  sem1).wait()
            process(j0 + 1, gbuf1)

            plsc.subcore_barrier()
            pltpu.sync_copy(
                yacc.at[pl.ds(base, rows_per_tile)],
                out_hbm.at[c, h, pl.ds(base, rows_per_tile)])

    return spmm


def _combine(a, b):
    def body(a_ref, b_ref, o_ref):
        o_ref[...] = a_ref[...] + b_ref[...]

    return pl.pallas_call(
        body, out_shape=jax.ShapeDtypeStruct(a.shape, a.dtype))(a, b)


def _dense(M, y1_halves, p20, p21, w0, w1, bias_col):
    C_out = w0.shape[0]
    CH = y1_halves.shape[2]

    def body(y1_ref, pa_ref, pb_ref, w0_ref, w1_ref, b_ref, o_ref):
        dn = (((1,), (1,)), ((), ()))
        acc = lax.dot_general(w0_ref[0, :, :CH], y1_ref[0],
                              dn, preferred_element_type=jnp.float32)
        acc += lax.dot_general(w0_ref[0, :, CH:], y1_ref[1],
                               dn, preferred_element_type=jnp.float32)
        y2_lo = pa_ref[0] + pb_ref[0]
        y2_hi = pa_ref[1] + pb_ref[1]
        acc += lax.dot_general(w1_ref[0, :, :CH], y2_lo,
                               dn, preferred_element_type=jnp.float32)
        acc += lax.dot_general(w1_ref[0, :, CH:], y2_hi,
                               dn, preferred_element_type=jnp.float32)
        o_ref[0] = acc[:, :M] + b_ref[...]

    return pl.pallas_call(
        body, out_shape=jax.ShapeDtypeStruct((1, C_out, M), jnp.float32),
    )(y1_halves, p20, p21, w0[None], w1[None], bias_col)


def kernel(x, edge_index, edge_values, theta, bias):
    _, C_in, M = x.shape
    E = edge_index.shape[1]
    per = _NW * _CHUNK
    nchunk = -(-E // per)
    nchunk += nchunk % 2
    E_pad = nchunk * per

    rows = edge_index[0]
    cols = edge_index[1]
    vals = edge_values
    if E_pad != E:
        rows = jnp.pad(rows, (0, E_pad - E))
        cols = jnp.pad(cols, (0, E_pad - E))
        vals = jnp.pad(vals, (0, E_pad - E))
    rows3 = rows.reshape(_NW, nchunk, _CHUNK)
    cols3 = cols.reshape(_NW, nchunk, _CHUNK)
    vals3 = vals.reshape(_NW, nchunk, _CHUNK)

    rpt = (-(-M // _NS) + 7) // 8 * 8
    M_pad = rpt * _NS
    CH = C_in // 2

    x_mc = x[0].T
    x0 = x_mc[:, :CH]
    x1 = x_mc[:, CH:]
    zinit = jnp.zeros((rpt, CH), jnp.float32)

    spmm = _make_spmm(M, M_pad, CH, nchunk)
    p1 = spmm(x0, x1, rows3, cols3, vals3, zinit)
    y1h = _combine(p1[0], p1[1])
    p2 = spmm(y1h[0], y1h[1], rows3, cols3, vals3, zinit)
    out = _dense(M, y1h, p2[0], p2[1], theta[:, :, 0], theta[:, :, 1],
                 bias[0])
    return out


# speedup vs baseline: 1.9528x; 1.9528x over previous
import functools

import jax
import jax.numpy as jnp
from jax import lax
from jax.experimental import pallas as pl
from jax.experimental.pallas import tpu as pltpu
from jax.experimental.pallas import tpu_sc as plsc

_NC = 2
_NS = 16
_CHUNK = 128
_LANES = 16


@functools.lru_cache(maxsize=None)
def _make_spmm(M, M_pad, CH, nchunk):
    rows_per_tile = M_pad // _NS

    mesh = plsc.VectorSubcoreMesh(core_axis_name="c", subcore_axis_name="s")

    @functools.partial(
        pl.kernel,
        out_type=jax.ShapeDtypeStruct((_NC, M_pad, CH), jnp.float32),
        mesh=mesh,
        scratch_types=[
            pltpu.VMEM((nchunk, _CHUNK), jnp.int32),
            pltpu.VMEM((nchunk, _CHUNK), jnp.int32),
            pltpu.VMEM((nchunk, _CHUNK), jnp.float32),
            pltpu.VMEM((_CHUNK, CH), jnp.float32),
            pltpu.VMEM((_CHUNK, CH), jnp.float32),
            pltpu.VMEM_SHARED((M_pad, CH), jnp.float32),
            pltpu.SemaphoreType.DMA,
            pltpu.SemaphoreType.DMA,
        ],
        compiler_params=pltpu.CompilerParams(use_tc_tiling_on_sc=False),
    )
    def spmm(x2_hbm, rows_hbm, cols_hbm, vals_hbm, zinit_hbm,
             out_hbm, rows_v, cols_v, vals_v, gbuf0, gbuf1, yacc,
             sem0, sem1):
        c = lax.axis_index("c")
        s = lax.axis_index("s")

        pltpu.sync_copy(rows_hbm.at[s], rows_v)
        pltpu.sync_copy(cols_hbm.at[s], cols_v)
        pltpu.sync_copy(vals_hbm.at[s], vals_v)
        base = pl.multiple_of(s * rows_per_tile, 8)

        pltpu.sync_copy(zinit_hbm, yacc.at[pl.ds(base, rows_per_tile)])
        plsc.subcore_barrier()

        def start(j, buf, sem):
            return pltpu.async_copy(x2_hbm.at[c].at[cols_v.at[j]], buf, sem)

        def wait(j, buf, sem):
            pltpu.make_async_copy(x2_hbm.at[c].at[cols_v.at[j]], buf,
                                  sem).wait()

        def process(j, buf):
            for eg in range(_CHUNK // _LANES):
                v16 = vals_v[j, pl.ds(eg * _LANES, _LANES)]
                for l in range(_LANES):
                    e = eg * _LANES + l
                    v = jnp.broadcast_to(v16[l], (_LANES,))
                    for g in range(CH // _LANES):
                        sl = pl.ds(g * _LANES, _LANES)
                        buf[e, sl] = buf[e, sl] * v
            pltpu.sync_copy(buf, yacc.at[rows_v.at[j]], add=True)

        assert nchunk % 2 == 0
        niter = nchunk // 2
        start(0, gbuf0, sem0)

        def pair_body(i, carry):
            j0 = 2 * i
            start(j0 + 1, gbuf1, sem1)
            wait(j0, gbuf0, sem0)
            process(j0, gbuf0)
            start(j0 + 2, gbuf0, sem0)
            wait(j0 + 1, gbuf1, sem1)
            process(j0 + 1, gbuf1)
            return carry

        lax.fori_loop(0, niter - 1, pair_body, 0)
        j0 = nchunk - 2
        start(j0 + 1, gbuf1, sem1)
        wait(j0, gbuf0, sem0)
        process(j0, gbuf0)
        wait(j0 + 1, gbuf1, sem1)
        process(j0 + 1, gbuf1)

        plsc.subcore_barrier()
        pltpu.sync_copy(
            yacc.at[pl.ds(base, rows_per_tile)],
            out_hbm.at[c, pl.ds(base, rows_per_tile)])

    return spmm


def _dense(M, y1h, y2h, w0, w1, bias_col):
    C_out = w0.shape[0]
    CH = y1h.shape[2]

    def body(y1_ref, y2_ref, w0_ref, w1_ref, b_ref, o_ref):
        dn = (((1,), (1,)), ((), ()))
        acc = lax.dot_general(w0_ref[0, :, :CH], y1_ref[0],
                              dn, preferred_element_type=jnp.float32)
        acc += lax.dot_general(w0_ref[0, :, CH:], y1_ref[1],
                               dn, preferred_element_type=jnp.float32)
        acc += lax.dot_general(w1_ref[0, :, :CH], y2_ref[0],
                               dn, preferred_element_type=jnp.float32)
        acc += lax.dot_general(w1_ref[0, :, CH:], y2_ref[1],
                               dn, preferred_element_type=jnp.float32)
        o_ref[0] = acc[:, :M] + b_ref[...]

    return pl.pallas_call(
        body, out_shape=jax.ShapeDtypeStruct((1, C_out, M), jnp.float32),
    )(y1h, y2h, w0[None], w1[None], bias_col)


def kernel(x, edge_index, edge_values, theta, bias):
    _, C_in, M = x.shape
    E = edge_index.shape[1]
    per = _NS * _CHUNK
    nchunk = -(-E // per)
    nchunk += nchunk % 2
    E_pad = nchunk * per

    rows = edge_index[0]
    cols = edge_index[1]
    vals = edge_values
    if E_pad != E:
        rows = jnp.pad(rows, (0, E_pad - E))
        cols = jnp.pad(cols, (0, E_pad - E))
        vals = jnp.pad(vals, (0, E_pad - E))
    rows3 = rows.reshape(_NS, nchunk, _CHUNK)
    cols3 = cols.reshape(_NS, nchunk, _CHUNK)
    vals3 = vals.reshape(_NS, nchunk, _CHUNK)

    rpt = (-(-M // _NS) + 7) // 8 * 8
    M_pad = rpt * _NS
    CH = C_in // 2

    x_mc = x[0].T
    x2 = jnp.stack([x_mc[:, :CH], x_mc[:, CH:]])
    zinit = jnp.zeros((rpt, CH), jnp.float32)

    spmm = _make_spmm(M, M_pad, CH, nchunk)
    y1h = spmm(x2, rows3, cols3, vals3, zinit)
    y2h = spmm(y1h, rows3, cols3, vals3, zinit)
    out = _dense(M, y1h, y2h, theta[:, :, 0], theta[:, :, 1], bias[0])
    return out


# speedup vs baseline: 2.4224x; 1.2405x over previous
import functools

import jax
import jax.numpy as jnp
from jax import lax
from jax.experimental import pallas as pl
from jax.experimental.pallas import tpu as pltpu
from jax.experimental.pallas import tpu_sc as plsc

_NC = 2
_NS = 16
_NQ = 4
_CHUNK = 128
_LANES = 16


@functools.lru_cache(maxsize=None)
def _make_spmm(M_pad, CQ, nchunk):
    rows_per_tile = M_pad // _NS

    mesh = plsc.VectorSubcoreMesh(core_axis_name="c", subcore_axis_name="s")

    @functools.partial(
        pl.kernel,
        out_type=jax.ShapeDtypeStruct((_NQ, M_pad, CQ), jnp.float32),
        mesh=mesh,
        scratch_types=[
            pltpu.VMEM((nchunk, _CHUNK), jnp.int32),
            pltpu.VMEM((nchunk, _CHUNK), jnp.int32),
            pltpu.VMEM((nchunk, _CHUNK), jnp.float32),
            pltpu.VMEM((_CHUNK, CQ), jnp.float32),
            pltpu.VMEM((_CHUNK, CQ), jnp.float32),
            pltpu.VMEM_SHARED((M_pad, CQ), jnp.float32),
            pltpu.VMEM_SHARED((M_pad, CQ), jnp.float32),
            pltpu.SemaphoreType.DMA,
            pltpu.SemaphoreType.DMA,
        ],
        compiler_params=pltpu.CompilerParams(use_tc_tiling_on_sc=False),
    )
    def spmm(x4_hbm, rows_hbm, cols_hbm, vals_hbm, zinit_hbm,
             out_hbm, rows_v, cols_v, vals_v, gbuf0, gbuf1, xsrc, yacc,
             sem0, sem1):
        c = lax.axis_index("c")
        s = lax.axis_index("s")

        pltpu.sync_copy(rows_hbm.at[s], rows_v)
        pltpu.sync_copy(cols_hbm.at[s], cols_v)
        pltpu.sync_copy(vals_hbm.at[s], vals_v)
        base = pl.multiple_of(s * rows_per_tile, 8)
        stripe = pl.ds(base, rows_per_tile)

        def start(j, buf, sem):
            return pltpu.async_copy(xsrc.at[cols_v.at[j]], buf, sem)

        def wait(j, buf, sem):
            pltpu.make_async_copy(xsrc.at[cols_v.at[j]], buf, sem).wait()

        def process(j, buf):
            for eg in range(_CHUNK // _LANES):
                v16 = vals_v[j, pl.ds(eg * _LANES, _LANES)]
                for l in range(_LANES):
                    e = eg * _LANES + l
                    v = jnp.broadcast_to(v16[l], (_LANES,))
                    for g in range(CQ // _LANES):
                        sl = pl.ds(g * _LANES, _LANES)
                        buf[e, sl] = buf[e, sl] * v
            pltpu.sync_copy(buf, yacc.at[rows_v.at[j]], add=True)

        assert nchunk % 2 == 0
        niter = nchunk // 2

        for q in range(2):
            qi = c * 2 + q
            pltpu.sync_copy(x4_hbm.at[qi, stripe], xsrc.at[stripe])
            pltpu.sync_copy(zinit_hbm, yacc.at[stripe])
            plsc.subcore_barrier()

            start(0, gbuf0, sem0)

            def pair_body(i, carry):
                j0 = 2 * i
                start(j0 + 1, gbuf1, sem1)
                wait(j0, gbuf0, sem0)
                process(j0, gbuf0)
                start(j0 + 2, gbuf0, sem0)
                wait(j0 + 1, gbuf1, sem1)
                process(j0 + 1, gbuf1)
                return carry

            lax.fori_loop(0, niter - 1, pair_body, 0)
            j0 = nchunk - 2
            start(j0 + 1, gbuf1, sem1)
            wait(j0, gbuf0, sem0)
            process(j0, gbuf0)
            wait(j0 + 1, gbuf1, sem1)
            process(j0 + 1, gbuf1)

            plsc.subcore_barrier()
            pltpu.sync_copy(yacc.at[stripe], out_hbm.at[qi, stripe])

    return spmm


def _dense(M, y1q, y2q, w0, w1, bias_col):
    C_out = w0.shape[0]
    CQ = y1q.shape[2]

    def body(y1_ref, y2_ref, w0_ref, w1_ref, b_ref, o_ref):
        dn = (((1,), (1,)), ((), ()))
        acc = None
        for q in range(_NQ):
            wsl = pl.ds(q * CQ, CQ)
            t = lax.dot_general(w0_ref[0, :, wsl], y1_ref[q], dn,
                                preferred_element_type=jnp.float32)
            t += lax.dot_general(w1_ref[0, :, wsl], y2_ref[q], dn,
                                 preferred_element_type=jnp.float32)
            acc = t if acc is None else acc + t
        o_ref[0] = acc[:, :M] + b_ref[...]

    return pl.pallas_call(
        body, out_shape=jax.ShapeDtypeStruct((1, C_out, M), jnp.float32),
    )(y1q, y2q, w0[None], w1[None], bias_col)


def kernel(x, edge_index, edge_values, theta, bias):
    _, C_in, M = x.shape
    E = edge_index.shape[1]
    per = _NS * _CHUNK
    nchunk = -(-E // per)
    nchunk += nchunk % 2
    E_pad = nchunk * per

    rows = edge_index[0]
    cols = edge_index[1]
    vals = edge_values
    if E_pad != E:
        rows = jnp.pad(rows, (0, E_pad - E))
        cols = jnp.pad(cols, (0, E_pad - E))
        vals = jnp.pad(vals, (0, E_pad - E))
    rows3 = rows.reshape(_NS, nchunk, _CHUNK)
    cols3 = cols.reshape(_NS, nchunk, _CHUNK)
    vals3 = vals.reshape(_NS, nchunk, _CHUNK)

    rpt = (-(-M // _NS) + 7) // 8 * 8
    M_pad = rpt * _NS
    CQ = C_in // _NQ

    x4 = jnp.pad(x[0].T, ((0, M_pad - M), (0, 0)))
    x4 = x4.reshape(M_pad, _NQ, CQ).transpose(1, 0, 2)
    zinit = jnp.zeros((rpt, CQ), jnp.float32)

    spmm = _make_spmm(M_pad, CQ, nchunk)
    y1q = spmm(x4, rows3, cols3, vals3, zinit)
    y2q = spmm(y1q, rows3, cols3, vals3, zinit)
    out = _dense(M, y1q, y2q, theta[:, :, 0], theta[:, :, 1], bias[0])
    return out


# speedup vs baseline: 2.5422x; 1.0495x over previous
import functools

import jax
import jax.numpy as jnp
from jax import lax
from jax.experimental import pallas as pl
from jax.experimental.pallas import tpu as pltpu
from jax.experimental.pallas import tpu_sc as plsc

_NC = 2
_NS = 16
_NQ = 4
_CHUNK = 128
_LANES = 16


@functools.lru_cache(maxsize=None)
def _make_spmm2(M_pad, CQ, nchunk):
    rows_per_tile = M_pad // _NS

    mesh = plsc.VectorSubcoreMesh(core_axis_name="c", subcore_axis_name="s")

    out_struct = jax.ShapeDtypeStruct((_NQ, M_pad, CQ), jnp.float32)

    @functools.partial(
        pl.kernel,
        out_type=(out_struct, out_struct),
        mesh=mesh,
        scratch_types=[
            pltpu.VMEM((nchunk, _CHUNK), jnp.int32),
            pltpu.VMEM((nchunk, _CHUNK), jnp.int32),
            pltpu.VMEM((nchunk, _CHUNK), jnp.float32),
            pltpu.VMEM((_CHUNK, CQ), jnp.float32),
            pltpu.VMEM((_CHUNK, CQ), jnp.float32),
            pltpu.VMEM((_CHUNK, CQ), jnp.float32),
            pltpu.VMEM((_CHUNK, CQ), jnp.float32),
            pltpu.VMEM_SHARED((M_pad, CQ), jnp.float32),
            pltpu.VMEM_SHARED((M_pad, CQ), jnp.float32),
            pltpu.SemaphoreType.DMA,
            pltpu.SemaphoreType.DMA,
            pltpu.SemaphoreType.DMA,
            pltpu.SemaphoreType.DMA,
        ],
        compiler_params=pltpu.CompilerParams(use_tc_tiling_on_sc=False),
    )
    def spmm2(x4_hbm, rows_hbm, cols_hbm, vals_hbm, zinit_hbm,
              y1_hbm, y2_hbm, rows_v, cols_v, vals_v, g0, g1, s0, s1,
              xsrc, yacc, sg0, sg1, ss0, ss1):
        c = lax.axis_index("c")
        s = lax.axis_index("s")

        pltpu.sync_copy(rows_hbm.at[s], rows_v)
        pltpu.sync_copy(cols_hbm.at[s], cols_v)
        pltpu.sync_copy(vals_hbm.at[s], vals_v)
        base = pl.multiple_of(s * rows_per_tile, 8)
        stripe = pl.ds(base, rows_per_tile)

        def g_start(src, j, buf, sem):
            pltpu.async_copy(src.at[cols_v.at[j]], buf, sem)

        def g_wait(src, j, buf, sem):
            pltpu.make_async_copy(src.at[cols_v.at[j]], buf, sem).wait()

        def s_start(dst, j, buf, sem):
            pltpu.async_copy(buf, dst.at[rows_v.at[j]], sem, add=True)

        def s_wait(dst, j, buf, sem):
            pltpu.make_async_copy(buf, dst.at[rows_v.at[j]], sem).wait()

        def scale(j, gb, sb):
            for eg in range(_CHUNK // _LANES):
                v16 = vals_v[j, pl.ds(eg * _LANES, _LANES)]
                for l in range(_LANES):
                    e = eg * _LANES + l
                    v = jnp.broadcast_to(v16[l], (_LANES,))
                    for g in range(CQ // _LANES):
                        sl = pl.ds(g * _LANES, _LANES)
                        sb[e, sl] = gb[e, sl] * v

        assert nchunk % 2 == 0 and nchunk >= 4
        niter = nchunk // 2

        def edge_pass(src, dst):
            g_start(src, 0, g0, sg0)
            g_start(src, 1, g1, sg1)
            g_wait(src, 0, g0, sg0)
            scale(0, g0, s0)
            s_start(dst, 0, s0, ss0)
            g_start(src, 2, g0, sg0)
            g_wait(src, 1, g1, sg1)
            scale(1, g1, s1)
            s_start(dst, 1, s1, ss1)
            g_start(src, 3, g1, sg1)

            def pair_body(i, carry):
                j0 = 2 * i
                g_wait(src, j0, g0, sg0)
                s_wait(dst, j0 - 2, s0, ss0)
                scale(j0, g0, s0)
                s_start(dst, j0, s0, ss0)
                g_start(src, j0 + 2, g0, sg0)
                g_wait(src, j0 + 1, g1, sg1)
                s_wait(dst, j0 - 1, s1, ss1)
                scale(j0 + 1, g1, s1)
                s_start(dst, j0 + 1, s1, ss1)
                g_start(src, j0 + 3, g1, sg1)
                return carry

            lax.fori_loop(1, niter - 1, pair_body, 0)
            j0 = nchunk - 2
            g_wait(src, j0, g0, sg0)
            s_wait(dst, j0 - 2, s0, ss0)
            scale(j0, g0, s0)
            s_start(dst, j0, s0, ss0)
            g_wait(src, j0 + 1, g1, sg1)
            s_wait(dst, j0 - 1, s1, ss1)
            scale(j0 + 1, g1, s1)
            s_start(dst, j0 + 1, s1, ss1)
            s_wait(dst, j0, s0, ss0)
            s_wait(dst, j0 + 1, s1, ss1)

        for q in range(2):
            qi = c * 2 + q
            pltpu.sync_copy(x4_hbm.at[qi, stripe], xsrc.at[stripe])
            pltpu.sync_copy(zinit_hbm, yacc.at[stripe])
            plsc.subcore_barrier()

            edge_pass(xsrc, yacc)

            plsc.subcore_barrier()
            pltpu.sync_copy(yacc.at[stripe], y1_hbm.at[qi, stripe])
            pltpu.sync_copy(zinit_hbm, xsrc.at[stripe])
            plsc.subcore_barrier()

            edge_pass(yacc, xsrc)

            plsc.subcore_barrier()
            pltpu.sync_copy(xsrc.at[stripe], y2_hbm.at[qi, stripe])

    return spmm2


def _dense(M, y1q, y2q, w0, w1, bias_col):
    C_out = w0.shape[0]
    CQ = y1q.shape[2]

    def body(y1_ref, y2_ref, w0_ref, w1_ref, b_ref, o_ref):
        dn = (((1,), (1,)), ((), ()))
        acc = None
        for q in range(_NQ):
            wsl = pl.ds(q * CQ, CQ)
            t = lax.dot_general(w0_ref[0, :, wsl], y1_ref[q], dn,
                                preferred_element_type=jnp.float32)
            t += lax.dot_general(w1_ref[0, :, wsl], y2_ref[q], dn,
                                 preferred_element_type=jnp.float32)
            acc = t if acc is None else acc + t
        o_ref[0] = acc[:, :M] + b_ref[...]

    return pl.pallas_call(
        body, out_shape=jax.ShapeDtypeStruct((1, C_out, M), jnp.float32),
    )(y1q, y2q, w0[None], w1[None], bias_col)


def kernel(x, edge_index, edge_values, theta, bias):
    _, C_in, M = x.shape
    E = edge_index.shape[1]
    per = _NS * _CHUNK
    nchunk = -(-E // per)
    nchunk += nchunk % 2
    E_pad = nchunk * per

    rows = edge_index[0]
    cols = edge_index[1]
    vals = edge_values
    if E_pad != E:
        rows = jnp.pad(rows, (0, E_pad - E))
        cols = jnp.pad(cols, (0, E_pad - E))
        vals = jnp.pad(vals, (0, E_pad - E))
    rows3 = rows.reshape(_NS, nchunk, _CHUNK)
    cols3 = cols.reshape(_NS, nchunk, _CHUNK)
    vals3 = vals.reshape(_NS, nchunk, _CHUNK)

    rpt = (-(-M // _NS) + 7) // 8 * 8
    M_pad = rpt * _NS
    CQ = C_in // _NQ

    x4 = jnp.pad(x[0].T, ((0, M_pad - M), (0, 0)))
    x4 = x4.reshape(M_pad, _NQ, CQ).transpose(1, 0, 2)
    zinit = jnp.zeros((rpt, CQ), jnp.float32)

    spmm2 = _make_spmm2(M_pad, CQ, nchunk)
    y1q, y2q = spmm2(x4, rows3, cols3, vals3, zinit)
    out = _dense(M, y1q, y2q, theta[:, :, 0], theta[:, :, 1], bias[0])
    return out
